# Initial kernel scaffold; baseline (speedup 1.0000x reference)
import jax
import jax.numpy as jnp
from jax.experimental import pallas as pl


def kernel(x, edge_attr, params, edge_index):
    raise NotImplementedError("write your pallas kernel here")


# speedup vs baseline: 2.0855x; 2.0855x over previous
import functools

import jax
import jax.numpy as jnp
from jax import lax
from jax.experimental import pallas as pl
from jax.experimental.pallas import tpu as pltpu
from jax.experimental.pallas import tpu_sc as plsc

_NC, _NS = 2, 16
_NW = _NC * _NS
_CH = 128
_EP = 81920


def _mesh():
    return plsc.VectorSubcoreMesh(core_axis_name="c", subcore_axis_name="s")


def _sc_gather(table, idx3d):
    _, D = table.shape
    cpw = idx3d.shape[1]
    ep = _NW * cpw * _CH

    @functools.partial(
        pl.kernel,
        out_type=jax.ShapeDtypeStruct((ep, D), jnp.float32),
        mesh=_mesh(),
        scratch_types=[
            pltpu.VMEM((cpw, _CH), jnp.int32),
            pltpu.VMEM((_CH, D), jnp.float32),
            pltpu.SemaphoreType.DMA,
        ],
    )
    def k(table_hbm, idx_hbm, out_hbm, idx_v, row_v, sem):
        wid = lax.axis_index("s") * _NC + lax.axis_index("c")
        pltpu.sync_copy(idx_hbm.at[wid], idx_v)

        def body(j, carry):
            pltpu.async_copy(table_hbm.at[idx_v.at[j]], row_v, sem).wait()
            pltpu.sync_copy(row_v, out_hbm.at[pl.ds((wid * cpw + j) * _CH, _CH)])
            return carry

        lax.fori_loop(0, cpw, body, 0)

    return k(table, idx3d)


def _sc_scatter_add(upd, idx3d, zeros_nd):
    _, D = upd.shape
    nrows = zeros_nd.shape[0]
    cpw = idx3d.shape[1]
    rps = nrows // _NS

    @functools.partial(
        pl.kernel,
        out_type=jax.ShapeDtypeStruct((_NC, nrows, D), jnp.float32),
        mesh=_mesh(),
        scratch_types=[
            pltpu.VMEM((cpw, _CH), jnp.int32),
            pltpu.VMEM((_CH, D), jnp.float32),
            pltpu.VMEM_SHARED((nrows, D), jnp.float32),
            pltpu.SemaphoreType.DMA,
        ],
    )
    def k(upd_hbm, idx_hbm, z_hbm, out_hbm, idx_v, buf_v, acc_sh, sem):
        cid = lax.axis_index("c")
        sid = lax.axis_index("s")
        wid = sid * _NC + cid
        r0 = sid * rps
        pltpu.sync_copy(z_hbm.at[pl.ds(r0, rps)], acc_sh.at[pl.ds(r0, rps)])
        pltpu.sync_copy(idx_hbm.at[wid], idx_v)
        plsc.subcore_barrier()

        def body(j, carry):
            pltpu.sync_copy(upd_hbm.at[pl.ds((wid * cpw + j) * _CH, _CH)], buf_v)
            pltpu.sync_copy(buf_v, acc_sh.at[idx_v.at[j]], add=True)
            return carry

        lax.fori_loop(0, cpw, body, 0)
        plsc.subcore_barrier()
        pltpu.sync_copy(acc_sh.at[pl.ds(r0, rps)], out_hbm.at[cid, pl.ds(r0, rps)])

    return k(upd, idx3d, zeros_nd)


def _edge_msg_body(ea_ref, xs_ref, w1_ref, b1_ref, w2_ref, b2r_ref, o_ref, *,
                   c, e_real, be):
    xsb = xs_ref[...][:, :c]
    hT = jnp.maximum(
        lax.dot_general(w1_ref[...], ea_ref[...], (((0,), (1,)), ((), ())),
                        preferred_element_type=jnp.float32) + b1_ref[...],
        0.0)
    Tt = lax.dot_general(w2_ref[...], xsb.astype(jnp.bfloat16),
                         (((1,), (1,)), ((), ())),
                         preferred_element_type=jnp.float32)
    acc = lax.dot_general(b2r_ref[...], xsb, (((0,), (1,)), ((), ())),
                          preferred_element_type=jnp.float32)
    for k in range(128):
        acc = acc + hT[k:k + 1, :] * Tt[k * c:(k + 1) * c, :]
    if c < 128:
        acc = jnp.concatenate([acc, jnp.zeros((128 - c, be), jnp.float32)], 0)
    msg = acc.T
    rows = lax.broadcasted_iota(jnp.int32, (be, 1), 0) + pl.program_id(0) * be
    o_ref[...] = jnp.where(rows < e_real, msg, 0.0)


def _tc_edge_msg(eap, xs, w1, b1col, w2catT_bf, b2r, e_real):
    ep = xs.shape[0]
    c = b2r.shape[0]
    be = 256
    body = functools.partial(_edge_msg_body, c=c, e_real=e_real, be=be)
    return pl.pallas_call(
        body,
        grid=(ep // be,),
        in_specs=[
            pl.BlockSpec((be, 4), lambda i: (i, 0)),
            pl.BlockSpec((be, 128), lambda i: (i, 0)),
            pl.BlockSpec((4, 128), lambda i: (0, 0)),
            pl.BlockSpec((128, 1), lambda i: (0, 0)),
            pl.BlockSpec((128 * c, c), lambda i: (0, 0)),
            pl.BlockSpec((c, c), lambda i: (0, 0)),
        ],
        out_specs=pl.BlockSpec((be, 128), lambda i: (i, 0)),
        out_shape=jax.ShapeDtypeStruct((ep, 128), jnp.float32),
    )(eap, xs, w1, b1col, w2catT_bf, b2r)


def _tc_linear(xa, w, brow):
    nn, ci = xa.shape
    co = w.shape[1]
    bn = 1000

    def body(x_ref, w_ref, b_ref, o_ref):
        o_ref[...] = jnp.dot(x_ref[...], w_ref[...],
                             preferred_element_type=jnp.float32) + b_ref[...]

    return pl.pallas_call(
        body,
        grid=(nn // bn,),
        in_specs=[
            pl.BlockSpec((bn, ci), lambda i: (i, 0)),
            pl.BlockSpec((ci, co), lambda i: (0, 0)),
            pl.BlockSpec((1, co), lambda i: (0, 0)),
        ],
        out_specs=pl.BlockSpec((bn, co), lambda i: (i, 0)),
        out_shape=jax.ShapeDtypeStruct((nn, co), jnp.float32),
    )(xa, w, brow)


def _tc_node_update(part2, cnt2, xp, root, biasrow, wp, bprow):
    nn = xp.shape[0]
    bn = 1000

    def body(p_ref, cnt_ref, xp_ref, root_ref, bias_ref, wp_ref, bp_ref, o_ref):
        agg = p_ref[0] + p_ref[1]
        cnt = cnt_ref[0, :, 0:1] + cnt_ref[1, :, 0:1]
        inv = 1.0 / jnp.maximum(cnt, 1.0)
        xpb = xp_ref[...]
        hres = jnp.maximum(
            agg * inv + jnp.dot(xpb, root_ref[...],
                                preferred_element_type=jnp.float32) + bias_ref[...],
            0.0) + xpb
        o_ref[...] = jnp.dot(hres, wp_ref[...],
                             preferred_element_type=jnp.float32) + bp_ref[...]

    return pl.pallas_call(
        body,
        grid=(nn // bn,),
        in_specs=[
            pl.BlockSpec((2, bn, 128), lambda i: (0, i, 0)),
            pl.BlockSpec((2, bn, 128), lambda i: (0, i, 0)),
            pl.BlockSpec((bn, 128), lambda i: (i, 0)),
            pl.BlockSpec((128, 128), lambda i: (0, 0)),
            pl.BlockSpec((1, 128), lambda i: (0, 0)),
            pl.BlockSpec((128, 128), lambda i: (0, 0)),
            pl.BlockSpec((1, 128), lambda i: (0, 0)),
        ],
        out_specs=pl.BlockSpec((bn, 128), lambda i: (i, 0)),
        out_shape=jax.ShapeDtypeStruct((nn, 128), jnp.float32),
    )(part2, cnt2, xp, root, biasrow, wp, bprow)


def _tc_edge_head(gr, gc, w2, b2):
    ep = gr.shape[0]
    c = w2.shape[0]
    be = 1024

    def body(a_ref, b_ref, w_ref, bb_ref, o_ref):
        ee = jnp.maximum(a_ref[...][:, :c] + b_ref[...][:, :c], 0.0)
        o_ref[...] = jnp.dot(ee, w_ref[...],
                             preferred_element_type=jnp.float32) + bb_ref[...]

    return pl.pallas_call(
        body,
        grid=(ep // be,),
        in_specs=[
            pl.BlockSpec((be, 128), lambda i: (i, 0)),
            pl.BlockSpec((be, 128), lambda i: (i, 0)),
            pl.BlockSpec((c, 1), lambda i: (0, 0)),
            pl.BlockSpec((1, 1), lambda i: (0, 0)),
        ],
        out_specs=pl.BlockSpec((be, 1), lambda i: (i, 0)),
        out_shape=jax.ShapeDtypeStruct((ep, 1), jnp.float32),
    )(gr, gc, w2, b2)


def kernel(x, edge_attr, params, edge_index):
    p = params
    nn = x.shape[0]
    e_real = edge_index.shape[1]

    src = edge_index[0].astype(jnp.int32)
    dst = edge_index[1].astype(jnp.int32)
    npad = ((nn + 127) // 128) * 128
    padi = jnp.arange(_EP - e_real, dtype=jnp.int32) % nn
    cpw = _EP // _CH // _NW
    src3 = jnp.concatenate([src, padi]).reshape(_NW, cpw, _CH)
    dst3 = jnp.concatenate([dst, padi]).reshape(_NW, cpw, _CH)
    eap = jnp.concatenate(
        [edge_attr, jnp.zeros((_EP - e_real, edge_attr.shape[1]), jnp.float32)])

    zeros_nd = jnp.zeros((npad, 128), jnp.float32)

    ones1 = jnp.zeros((_EP, 128), jnp.float32).at[:e_real, 0].set(1.0)
    cnt2 = _sc_scatter_add(ones1, dst3, zeros_nd)

    def pad2(w, r=128, co=128):
        out = jnp.zeros((r, co), jnp.float32)
        return out.at[:w.shape[0], :w.shape[1]].set(w)

    xp = _tc_linear(x, p['proj1'][0], p['proj1'][1].reshape(1, -1))

    layers = [
        (128, 'nn1_l1', 'nn1_l2', 'conv1_root', 'conv1_bias', 'proj2'),
        (64, 'nn2_l1', 'nn2_l2', 'conv2_root', 'conv2_bias', 'proj3'),
        (32, 'nn3_l1', 'nn3_l2', 'conv3_root', 'conv3_bias', 'proj4'),
        (16, 'nn4_l1', 'nn4_l2', 'conv4_root', 'conv4_bias', None),
    ]
    for c, n1, n2, rt, bs, pj in layers:
        w1, b1 = p[n1]
        w2, b2 = p[n2]
        w2catT = w2.reshape(128, c, c).transpose(0, 2, 1).reshape(128 * c, c)
        w2catT = w2catT.astype(jnp.bfloat16)
        b2r = b2.reshape(c, c)

        xs = _sc_gather(xp, src3)
        msg = _tc_edge_msg(eap, xs, w1, b1.reshape(128, 1), w2catT, b2r, e_real)
        part2 = _sc_scatter_add(msg, dst3, zeros_nd)

        if pj is not None:
            wp, bp = p[pj][0], p[pj][1]
        else:
            wp, bp = p['edge_lin1'][0], 0.5 * p['edge_lin1'][1]
        xp = _tc_node_update(part2, cnt2, xp, pad2(p[rt]),
                             pad2(p[bs].reshape(1, -1), 1),
                             pad2(wp), pad2(bp.reshape(1, -1), 1))

    both = _sc_gather(xp, jnp.concatenate([src3, dst3], axis=1))
    both4 = both.reshape(_NW, 2, cpw * _CH, 128)
    gr = both4[:, 0].reshape(_EP, 128)
    gc = both4[:, 1].reshape(_EP, 128)
    out2 = _tc_edge_head(gr, gc,
                         p['edge_lin2'][0], p['edge_lin2'][1].reshape(1, 1))
    return out2[:e_real, 0]
